# baseline (speedup 1.0000x reference)
import numpy as np

import jax
import jax.numpy as jnp
from jax import lax
from jax.experimental import pallas as pl
from jax.experimental.pallas import tpu as pltpu
from jax.experimental.pallas import tpu_sc as plsc

_NB = 10
_N = 16777216
_NC = 2
_NS = 16
_NW = _NC * _NS
_L = 16
_PER_W = _N // _NW
_C = 16384
_NCHUNK = _PER_W // _C
_ACC = _NB * _L
_U = 8

_TBL_N = 2048
_TBL_LO = -16.0
_TBL_SCALE = 64.0
_xs = _TBL_LO + np.arange(_TBL_N + 1, dtype=np.float64) / _TBL_SCALE
_sig = 1.0 / (1.0 + np.exp(-_xs))
_TBL_BASE = np.asarray(_sig[:-1], dtype=np.float32)
_TBL_SLOPE = np.asarray(_sig[1:] - _sig[:-1], dtype=np.float32)


def _pass_a_body(logits_hbm, tb_hbm, ts_hbm, outf_hbm, bins_hbm,
                 lbuf0, lbuf1, obuf0, obuf1, tb, ts,
                 acc_cnt, acc_conf, sem0, sem1, semo0, semo1):
    i32 = jnp.int32
    wid = lax.axis_index("s") * i32(_NC) + lax.axis_index("c")
    base = wid * i32(_PER_W)

    pltpu.sync_copy(tb_hbm, tb)
    pltpu.sync_copy(ts_hbm, ts)

    zf = jnp.zeros((_L,), jnp.float32)
    for k in range(_NB):
        acc_cnt[pl.ds(k * _L, _L)] = zf
        acc_conf[pl.ds(k * _L, _L)] = zf

    def start(i, lbuf, sem):
        off = base + i * i32(_C)
        pltpu.async_copy(logits_hbm.at[pl.ds(off, _C)], lbuf, sem)

    def wait_in(lbuf, sem):
        pltpu.make_async_copy(logits_hbm.at[pl.ds(0, _C)], lbuf, sem).wait()

    def start_out(i, obuf, semo):
        off = base + i * i32(_C)
        pltpu.async_copy(obuf, bins_hbm.at[pl.ds(off, _C)], semo)

    def wait_out(obuf, semo):
        pltpu.make_async_copy(obuf, bins_hbm.at[pl.ds(0, _C)], semo).wait()

    start(0, lbuf0, sem0)
    start(1, lbuf1, sem1)

    lane = lax.iota(jnp.int32, _L)
    ones = jnp.ones((_L,), jnp.float32)

    def consume(lbuf, obuf):
        def inner(j, carry):
            base_e = j * i32(_U * _L)
            offs = [base_e + i32(u * _L) for u in range(_U)]
            xs = [lbuf[pl.ds(o, _L)] for o in offs]
            tts = [x * _TBL_SCALE + (-_TBL_LO * _TBL_SCALE) for x in xs]
            tts = [jnp.minimum(jnp.maximum(t, 0.0), _TBL_N - 0.004)
                   for t in tts]
            iis = [t.astype(jnp.int32) for t in tts]
            fracs = [t - i.astype(jnp.float32) for t, i in zip(tts, iis)]
            bas = [plsc.load_gather(tb, [i]) for i in iis]
            sls = [plsc.load_gather(ts, [i]) for i in iis]
            confs = [b + f * s for b, f, s in zip(bas, fracs, sls)]
            bis = [(c * 10.0).astype(jnp.int32) for c in confs]
            addrs = [b * i32(_L) + lane for b in bis]
            for u in range(_U):
                plsc.addupdate_scatter(acc_cnt, [addrs[u]], ones)
                plsc.addupdate_scatter(acc_conf, [addrs[u]], confs[u])
            for u in range(_U):
                obuf[pl.ds(offs[u], _L)] = bis[u]
            return carry
        lax.fori_loop(i32(0), i32(_C // (_U * _L)), inner, i32(0))

    def outer(t, carry):
        i0 = t * i32(2)

        wait_in(lbuf0, sem0)

        @pl.when(i0 >= i32(2))
        def _():
            wait_out(obuf0, semo0)

        consume(lbuf0, obuf0)
        start_out(i0, obuf0, semo0)

        @pl.when(i0 + i32(2) < i32(_NCHUNK))
        def _():
            start(i0 + i32(2), lbuf0, sem0)

        wait_in(lbuf1, sem1)

        @pl.when(i0 >= i32(2))
        def _():
            wait_out(obuf1, semo1)

        consume(lbuf1, obuf1)
        start_out(i0 + i32(1), obuf1, semo1)

        @pl.when(i0 + i32(3) < i32(_NCHUNK))
        def _():
            start(i0 + i32(3), lbuf1, sem1)

        return carry

    lax.fori_loop(i32(0), i32(_NCHUNK // 2), outer, i32(0))

    wait_out(obuf0, semo0)
    wait_out(obuf1, semo1)

    obase = wid * i32(2 * _ACC)
    pltpu.sync_copy(acc_cnt, outf_hbm.at[pl.ds(obase, _ACC)])
    pltpu.sync_copy(acc_conf, outf_hbm.at[pl.ds(obase + i32(_ACC), _ACC)])


_KB = 4
_PER_WB = _PER_W // _KB
_NCHUNK_B = _PER_WB // _C


def _pass_b_body(labels_hbm, bins_hbm, outi_hbm,
                 bbuf0, bbuf1, qbuf0, qbuf1, acc_lab, sem0, sem1):
    i32 = jnp.int32
    wid = lax.axis_index("s") * i32(_NC) + lax.axis_index("c")
    base = wid * i32(_PER_WB)

    zi = jnp.zeros((_L,), jnp.int32)
    for k in range(_NB):
        acc_lab[pl.ds(k * _L, _L)] = zi

    def start(i, bbuf, qbuf, sem):
        off = base + i * i32(_C)
        pltpu.async_copy(labels_hbm.at[pl.ds(off, _C)], bbuf, sem)
        pltpu.async_copy(bins_hbm.at[pl.ds(off, _C)], qbuf, sem)

    def wait(bbuf, qbuf, sem):
        pltpu.make_async_copy(labels_hbm.at[pl.ds(0, _C)], bbuf, sem).wait()
        pltpu.make_async_copy(bins_hbm.at[pl.ds(0, _C)], qbuf, sem).wait()

    start(0, bbuf0, qbuf0, sem0)
    start(1, bbuf1, qbuf1, sem1)

    lane = lax.iota(jnp.int32, _L)

    def consume(bbuf, qbuf):
        def inner(j, carry):
            base_e = j * i32(_U * _L)
            offs = [base_e + i32(u * _L) for u in range(_U)]
            labs = [plsc.bitcast(bbuf[pl.ds(o, _L)], jnp.int32)
                    for o in offs]
            bis = [qbuf[pl.ds(o, _L)] for o in offs]
            addrs = [b * i32(_L) + lane for b in bis]
            for u in range(_U):
                plsc.addupdate_scatter(acc_lab, [addrs[u]], labs[u])
            return carry
        lax.fori_loop(i32(0), i32(_C // (_U * _L)), inner, i32(0))

    def outer(t, carry):
        i0 = t * i32(2)
        wait(bbuf0, qbuf0, sem0)
        consume(bbuf0, qbuf0)

        @pl.when(i0 + i32(2) < i32(_NCHUNK_B))
        def _():
            start(i0 + i32(2), bbuf0, qbuf0, sem0)

        wait(bbuf1, qbuf1, sem1)
        consume(bbuf1, qbuf1)

        @pl.when(i0 + i32(3) < i32(_NCHUNK_B))
        def _():
            start(i0 + i32(3), bbuf1, qbuf1, sem1)

        return carry

    lax.fori_loop(i32(0), i32(_NCHUNK_B // 2), outer, i32(0))

    pltpu.sync_copy(acc_lab, outi_hbm.at[pl.ds(wid * i32(_ACC), _ACC)])


_MESH = plsc.VectorSubcoreMesh(core_axis_name="c", subcore_axis_name="s",
                               num_cores=_NC, num_subcores=_NS)


@jax.jit
def _sc_run(logits, labels, tbl_base, tbl_slope):
    pass_a = pl.kernel(
        _pass_a_body,
        out_type=(
            jax.ShapeDtypeStruct((_NW * 2 * _ACC,), jnp.float32),
            jax.ShapeDtypeStruct((_N,), jnp.int32),
        ),
        mesh=_MESH,
        scratch_types=[
            pltpu.VMEM((_C,), jnp.float32),
            pltpu.VMEM((_C,), jnp.float32),
            pltpu.VMEM((_C,), jnp.int32),
            pltpu.VMEM((_C,), jnp.int32),
            pltpu.VMEM((_TBL_N,), jnp.float32),
            pltpu.VMEM((_TBL_N,), jnp.float32),
            pltpu.VMEM((_ACC,), jnp.float32),
            pltpu.VMEM((_ACC,), jnp.float32),
            pltpu.SemaphoreType.DMA,
            pltpu.SemaphoreType.DMA,
            pltpu.SemaphoreType.DMA,
            pltpu.SemaphoreType.DMA,
        ],
        compiler_params=pltpu.CompilerParams(needs_layout_passes=False),
    )
    outf, bins = pass_a(logits, tbl_base, tbl_slope)

    pass_b = pl.kernel(
        _pass_b_body,
        out_type=jax.ShapeDtypeStruct((_NW * _ACC,), jnp.int32),
        mesh=_MESH,
        scratch_types=[
            pltpu.VMEM((_C,), jnp.uint32),
            pltpu.VMEM((_C,), jnp.uint32),
            pltpu.VMEM((_C,), jnp.int32),
            pltpu.VMEM((_C,), jnp.int32),
            pltpu.VMEM((_ACC,), jnp.int32),
            pltpu.SemaphoreType.DMA,
            pltpu.SemaphoreType.DMA,
        ],
        compiler_params=pltpu.CompilerParams(needs_layout_passes=False),
    )
    nk = _N // _KB
    outis = [
        pass_b(
            lax.convert_element_type(
                lax.optimization_barrier(
                    lax.slice(labels, (k * nk,), ((k + 1) * nk,))),
                jnp.uint32),
            lax.slice(bins, (k * nk,), ((k + 1) * nk,)))
        for k in range(_KB)
    ]
    return outf, jnp.concatenate(outis)


def _combine_body(pf_ref, pi_ref, pc_ref, e_ref, m_ref):
    xf = pf_ref[...]
    xi = pi_ref[...]
    sf = jnp.sum(xf, axis=(0, 3))
    lab = jnp.sum(xi.astype(jnp.float32), axis=(0, 2))
    cnt = sf[0]
    cf = sf[1]
    nonempty = cnt > 0.0
    denom = jnp.maximum(cnt, 1.0)
    pos = jnp.where(nonempty, lab / denom, 0.0)
    cfm = jnp.where(nonempty, cf / denom, 0.0)
    ece_i = jnp.abs(pos - cfm)
    pc_ref[...] = jnp.stack([pos, cfm])
    e_ref[...] = jnp.sum(ece_i).reshape(1, 1)
    m_ref[...] = jnp.max(ece_i).reshape(1, 1)


def kernel(logits, labels):
    if labels.dtype != jnp.int64:
        labels = labels.astype(jnp.int64)
    partials_f, partials_i = _sc_run(
        logits, labels, jnp.asarray(_TBL_BASE), jnp.asarray(_TBL_SLOPE))
    pf = partials_f.reshape(_NW, 2, _NB, _L)
    pi = partials_i.reshape(_KB * _NW, _NB, _L)
    pc, e, m = pl.pallas_call(
        _combine_body,
        out_shape=[
            jax.ShapeDtypeStruct((2, _NB), jnp.float32),
            jax.ShapeDtypeStruct((1, 1), jnp.float32),
            jax.ShapeDtypeStruct((1, 1), jnp.float32),
        ],
    )(pf, pi)
    return (pc[0], pc[1], e[0, 0], m[0, 0])


# speedup vs baseline: 1.1683x; 1.1683x over previous
import numpy as np

import jax
import jax.numpy as jnp
from jax import lax
from jax.experimental import pallas as pl
from jax.experimental.pallas import tpu as pltpu
from jax.experimental.pallas import tpu_sc as plsc

_NB = 10
_N = 16777216
_NC = 2
_NS = 16
_NW = _NC * _NS
_L = 16
_PER_W = _N // _NW
_C = 16384
_NCHUNK = _PER_W // _C
_ACC = _NB * _L
_U = 8

_TBL_N = 2048
_TBL_LO = -16.0
_TBL_SCALE = 64.0
_xs = _TBL_LO + np.arange(_TBL_N + 1, dtype=np.float64) / _TBL_SCALE
_sig = 1.0 / (1.0 + np.exp(-_xs))
_TBL_BASE = np.asarray(_sig[:-1], dtype=np.float32)
_TBL_SLOPE = np.asarray(_sig[1:] - _sig[:-1], dtype=np.float32)


def _pass_a_body(logits_hbm, tb_hbm, ts_hbm, outf_hbm, bins_hbm,
                 lbuf0, lbuf1, obuf0, obuf1, tb, ts,
                 acc_cnt, acc_conf, sem0, sem1, semo0, semo1):
    i32 = jnp.int32
    wid = lax.axis_index("s") * i32(_NC) + lax.axis_index("c")
    base = wid * i32(_PER_W)

    pltpu.sync_copy(tb_hbm, tb)
    pltpu.sync_copy(ts_hbm, ts)

    zf = jnp.zeros((_L,), jnp.float32)
    for k in range(_NB):
        acc_cnt[pl.ds(k * _L, _L)] = zf
        acc_conf[pl.ds(k * _L, _L)] = zf

    def start(i, lbuf, sem):
        off = base + i * i32(_C)
        pltpu.async_copy(logits_hbm.at[pl.ds(off, _C)], lbuf, sem)

    def wait_in(lbuf, sem):
        pltpu.make_async_copy(logits_hbm.at[pl.ds(0, _C)], lbuf, sem).wait()

    def start_out(i, obuf, semo):
        off = base + i * i32(_C)
        pltpu.async_copy(obuf, bins_hbm.at[pl.ds(off, _C)], semo)

    def wait_out(obuf, semo):
        pltpu.make_async_copy(obuf, bins_hbm.at[pl.ds(0, _C)], semo).wait()

    start(0, lbuf0, sem0)
    start(1, lbuf1, sem1)

    lane = lax.iota(jnp.int32, _L)
    ones = jnp.ones((_L,), jnp.float32)

    def consume(lbuf, obuf):
        def inner(j, carry):
            base_e = j * i32(_U * _L)
            offs = [base_e + i32(u * _L) for u in range(_U)]
            xs = [lbuf[pl.ds(o, _L)] for o in offs]
            tts = [x * _TBL_SCALE + (-_TBL_LO * _TBL_SCALE) for x in xs]
            tts = [jnp.minimum(jnp.maximum(t, 0.0), _TBL_N - 0.004)
                   for t in tts]
            iis = [t.astype(jnp.int32) for t in tts]
            fracs = [t - i.astype(jnp.float32) for t, i in zip(tts, iis)]
            bas = [plsc.load_gather(tb, [i]) for i in iis]
            sls = [plsc.load_gather(ts, [i]) for i in iis]
            confs = [b + f * s for b, f, s in zip(bas, fracs, sls)]
            bis = [(c * 10.0).astype(jnp.int32) for c in confs]
            addrs = [b * i32(_L) + lane for b in bis]
            for u in range(_U):
                plsc.addupdate_scatter(acc_cnt, [addrs[u]], ones)
                plsc.addupdate_scatter(acc_conf, [addrs[u]], confs[u])
            for u in range(_U):
                obuf[pl.ds(offs[u], _L)] = addrs[u]
            return carry
        lax.fori_loop(i32(0), i32(_C // (_U * _L)), inner, i32(0))

    def outer(t, carry):
        i0 = t * i32(2)

        wait_in(lbuf0, sem0)

        @pl.when(i0 >= i32(2))
        def _():
            wait_out(obuf0, semo0)

        consume(lbuf0, obuf0)
        start_out(i0, obuf0, semo0)

        @pl.when(i0 + i32(2) < i32(_NCHUNK))
        def _():
            start(i0 + i32(2), lbuf0, sem0)

        wait_in(lbuf1, sem1)

        @pl.when(i0 >= i32(2))
        def _():
            wait_out(obuf1, semo1)

        consume(lbuf1, obuf1)
        start_out(i0 + i32(1), obuf1, semo1)

        @pl.when(i0 + i32(3) < i32(_NCHUNK))
        def _():
            start(i0 + i32(3), lbuf1, sem1)

        return carry

    lax.fori_loop(i32(0), i32(_NCHUNK // 2), outer, i32(0))

    wait_out(obuf0, semo0)
    wait_out(obuf1, semo1)

    obase = wid * i32(2 * _ACC)
    pltpu.sync_copy(acc_cnt, outf_hbm.at[pl.ds(obase, _ACC)])
    pltpu.sync_copy(acc_conf, outf_hbm.at[pl.ds(obase + i32(_ACC), _ACC)])


def _pass_b_body(labels_hbm, bins_hbm, outi_hbm,
                 bbuf0, bbuf1, qbuf0, qbuf1, acc_lab, sem0, sem1):
    i32 = jnp.int32
    wid = lax.axis_index("s") * i32(_NC) + lax.axis_index("c")
    base = wid * i32(_PER_W)

    zi = jnp.zeros((_L,), jnp.int32)
    for k in range(_NB):
        acc_lab[pl.ds(k * _L, _L)] = zi

    def start(i, bbuf, qbuf, sem):
        off = base + i * i32(_C)
        pltpu.async_copy(labels_hbm.at[pl.ds(off, _C)], bbuf, sem)
        pltpu.async_copy(bins_hbm.at[pl.ds(off, _C)], qbuf, sem)

    def wait(bbuf, qbuf, sem):
        pltpu.make_async_copy(labels_hbm.at[pl.ds(0, _C)], bbuf, sem).wait()
        pltpu.make_async_copy(bins_hbm.at[pl.ds(0, _C)], qbuf, sem).wait()

    start(0, bbuf0, qbuf0, sem0)
    start(1, bbuf1, qbuf1, sem1)

    lane = lax.iota(jnp.int32, _L)

    def consume(bbuf, qbuf):
        def inner(j, carry):
            base_e = j * i32(_U * _L)
            offs = [base_e + i32(u * _L) for u in range(_U)]
            labs = [plsc.bitcast(bbuf[pl.ds(o, _L)], jnp.int32)
                    for o in offs]
            addrs = [qbuf[pl.ds(o, _L)] for o in offs]
            for u in range(_U):
                plsc.addupdate_scatter(acc_lab, [addrs[u]], labs[u])
            return carry
        lax.fori_loop(i32(0), i32(_C // (_U * _L)), inner, i32(0))

    def outer(t, carry):
        i0 = t * i32(2)
        wait(bbuf0, qbuf0, sem0)
        consume(bbuf0, qbuf0)

        @pl.when(i0 + i32(2) < i32(_NCHUNK))
        def _():
            start(i0 + i32(2), bbuf0, qbuf0, sem0)

        wait(bbuf1, qbuf1, sem1)
        consume(bbuf1, qbuf1)

        @pl.when(i0 + i32(3) < i32(_NCHUNK))
        def _():
            start(i0 + i32(3), bbuf1, qbuf1, sem1)

        return carry

    lax.fori_loop(i32(0), i32(_NCHUNK // 2), outer, i32(0))

    pltpu.sync_copy(acc_lab, outi_hbm.at[pl.ds(wid * i32(_ACC), _ACC)])


_MESH = plsc.VectorSubcoreMesh(core_axis_name="c", subcore_axis_name="s",
                               num_cores=_NC, num_subcores=_NS)


@jax.jit
def _sc_run(logits, labels_u32, tbl_base, tbl_slope):
    pass_a = pl.kernel(
        _pass_a_body,
        out_type=(
            jax.ShapeDtypeStruct((_NW * 2 * _ACC,), jnp.float32),
            jax.ShapeDtypeStruct((_N,), jnp.int32),
        ),
        mesh=_MESH,
        scratch_types=[
            pltpu.VMEM((_C,), jnp.float32),
            pltpu.VMEM((_C,), jnp.float32),
            pltpu.VMEM((_C,), jnp.int32),
            pltpu.VMEM((_C,), jnp.int32),
            pltpu.VMEM((_TBL_N,), jnp.float32),
            pltpu.VMEM((_TBL_N,), jnp.float32),
            pltpu.VMEM((_ACC,), jnp.float32),
            pltpu.VMEM((_ACC,), jnp.float32),
            pltpu.SemaphoreType.DMA,
            pltpu.SemaphoreType.DMA,
            pltpu.SemaphoreType.DMA,
            pltpu.SemaphoreType.DMA,
        ],
        compiler_params=pltpu.CompilerParams(needs_layout_passes=False),
    )
    outf, bins = pass_a(logits, tbl_base, tbl_slope)

    pass_b = pl.kernel(
        _pass_b_body,
        out_type=jax.ShapeDtypeStruct((_NW * _ACC,), jnp.int32),
        mesh=_MESH,
        scratch_types=[
            pltpu.VMEM((_C,), jnp.uint32),
            pltpu.VMEM((_C,), jnp.uint32),
            pltpu.VMEM((_C,), jnp.int32),
            pltpu.VMEM((_C,), jnp.int32),
            pltpu.VMEM((_ACC,), jnp.int32),
            pltpu.SemaphoreType.DMA,
            pltpu.SemaphoreType.DMA,
        ],
        compiler_params=pltpu.CompilerParams(needs_layout_passes=False),
    )
    outi = pass_b(labels_u32, bins)
    return outf, outi


def _combine_body(pf_ref, pi_ref, pc_ref, e_ref, m_ref):
    xf = pf_ref[...]
    xi = pi_ref[...]
    sf = jnp.sum(xf, axis=(0, 3))
    lab = jnp.sum(xi.astype(jnp.float32), axis=(0, 2))
    cnt = sf[0]
    cf = sf[1]
    nonempty = cnt > 0.0
    denom = jnp.maximum(cnt, 1.0)
    pos = jnp.where(nonempty, lab / denom, 0.0)
    cfm = jnp.where(nonempty, cf / denom, 0.0)
    ece_i = jnp.abs(pos - cfm)
    pc_ref[...] = jnp.stack([pos, cfm])
    e_ref[...] = jnp.sum(ece_i).reshape(1, 1)
    m_ref[...] = jnp.max(ece_i).reshape(1, 1)


def kernel(logits, labels):
    labels_u32 = labels.astype(jnp.uint32)
    partials_f, partials_i = _sc_run(
        logits, labels_u32, jnp.asarray(_TBL_BASE), jnp.asarray(_TBL_SLOPE))
    pf = partials_f.reshape(_NW, 2, _NB, _L)
    pi = partials_i.reshape(_NW, _NB, _L)
    pc, e, m = pl.pallas_call(
        _combine_body,
        out_shape=[
            jax.ShapeDtypeStruct((2, _NB), jnp.float32),
            jax.ShapeDtypeStruct((1, 1), jnp.float32),
            jax.ShapeDtypeStruct((1, 1), jnp.float32),
        ],
    )(pf, pi)
    return (pc[0], pc[1], e[0, 0], m[0, 0])


# speedup vs baseline: 1.2016x; 1.0285x over previous
import numpy as np

import jax
import jax.numpy as jnp
from jax import lax
from jax.experimental import pallas as pl
from jax.experimental.pallas import tpu as pltpu
from jax.experimental.pallas import tpu_sc as plsc

_NB = 10
_N = 16777216
_NC = 2
_NS = 16
_NW = _NC * _NS
_L = 16
_PER_W = _N // _NW
_C = 16384
_NCHUNK = _PER_W // _C
_ACC = _NB * _L
_U = 8

_TBL_N = 2048
_TBL_LO = -16.0
_TBL_SCALE = 64.0
_xs = _TBL_LO + np.arange(_TBL_N + 1, dtype=np.float64) / _TBL_SCALE
_sig = 1.0 / (1.0 + np.exp(-_xs))
_TBL_BASE = np.asarray(_sig[:-1], dtype=np.float32)
_TBL_SLOPE = np.asarray(_sig[1:] - _sig[:-1], dtype=np.float32)


def _pass_a_body(logits_hbm, tb_hbm, ts_hbm, outf_hbm, bins_hbm,
                 lbuf0, lbuf1, obuf0, obuf1, tb, ts,
                 acc_cnt, acc_conf, sem0, sem1, semo0, semo1):
    i32 = jnp.int32
    wid = lax.axis_index("s") * i32(_NC) + lax.axis_index("c")
    base = wid * i32(_PER_W)

    pltpu.sync_copy(tb_hbm, tb)
    pltpu.sync_copy(ts_hbm, ts)

    zf = jnp.zeros((_L,), jnp.float32)
    for k in range(_NB):
        acc_cnt[pl.ds(k * _L, _L)] = zf
        acc_conf[pl.ds(k * _L, _L)] = zf

    def start(i, lbuf, sem):
        off = base + i * i32(_C)
        pltpu.async_copy(logits_hbm.at[pl.ds(off, _C)], lbuf, sem)

    def wait_in(lbuf, sem):
        pltpu.make_async_copy(logits_hbm.at[pl.ds(0, _C)], lbuf, sem).wait()

    def start_out(i, obuf, semo):
        off = wid * i32(_PER_W // 2) + i * i32(_C // 2)
        pltpu.async_copy(obuf, bins_hbm.at[pl.ds(off, _C // 2)], semo)

    def wait_out(obuf, semo):
        pltpu.make_async_copy(obuf, bins_hbm.at[pl.ds(0, _C // 2)],
                              semo).wait()

    start(0, lbuf0, sem0)
    start(1, lbuf1, sem1)

    lane = lax.iota(jnp.int32, _L)
    ones = jnp.ones((_L,), jnp.float32)

    def consume(lbuf, obuf):
        def inner(j, carry):
            base_e = j * i32(_U * _L)
            offs = [base_e + i32(u * _L) for u in range(_U)]
            xs = [lbuf[pl.ds(o, _L)] for o in offs]
            tts = [x * _TBL_SCALE + (-_TBL_LO * _TBL_SCALE) for x in xs]
            tts = [jnp.minimum(jnp.maximum(t, 0.0), _TBL_N - 0.004)
                   for t in tts]
            iis = [t.astype(jnp.int32) for t in tts]
            fracs = [t - i.astype(jnp.float32) for t, i in zip(tts, iis)]
            bas = [plsc.load_gather(tb, [i]) for i in iis]
            sls = [plsc.load_gather(ts, [i]) for i in iis]
            confs = [b + f * s for b, f, s in zip(bas, fracs, sls)]
            bis = [(c * 10.0).astype(jnp.int32) for c in confs]
            addrs = [b * i32(_L) + lane for b in bis]
            for u in range(_U):
                plsc.addupdate_scatter(acc_cnt, [addrs[u]], ones)
                plsc.addupdate_scatter(acc_conf, [addrs[u]], confs[u])
            base_h = j * i32(_U * _L // 2)
            for g in range(_U // 2):
                p = plsc.pack(addrs[2 * g], addrs[2 * g + 1],
                              format=plsc.PackFormat.INTERLEAVED)
                obuf[pl.ds(base_h + i32(g * _L), _L)] = plsc.bitcast(
                    p, jnp.int32)
            return carry
        lax.fori_loop(i32(0), i32(_C // (_U * _L)), inner, i32(0))

    def outer(t, carry):
        i0 = t * i32(2)

        wait_in(lbuf0, sem0)

        @pl.when(i0 >= i32(2))
        def _():
            wait_out(obuf0, semo0)

        consume(lbuf0, obuf0)
        start_out(i0, obuf0, semo0)

        @pl.when(i0 + i32(2) < i32(_NCHUNK))
        def _():
            start(i0 + i32(2), lbuf0, sem0)

        wait_in(lbuf1, sem1)

        @pl.when(i0 >= i32(2))
        def _():
            wait_out(obuf1, semo1)

        consume(lbuf1, obuf1)
        start_out(i0 + i32(1), obuf1, semo1)

        @pl.when(i0 + i32(3) < i32(_NCHUNK))
        def _():
            start(i0 + i32(3), lbuf1, sem1)

        return carry

    lax.fori_loop(i32(0), i32(_NCHUNK // 2), outer, i32(0))

    wait_out(obuf0, semo0)
    wait_out(obuf1, semo1)

    obase = wid * i32(2 * _ACC)
    pltpu.sync_copy(acc_cnt, outf_hbm.at[pl.ds(obase, _ACC)])
    pltpu.sync_copy(acc_conf, outf_hbm.at[pl.ds(obase + i32(_ACC), _ACC)])


def _pass_b_body(labels_hbm, bins_hbm, outi_hbm,
                 bbuf0, bbuf1, qbuf0, qbuf1, acc_lab, sem0, sem1):
    i32 = jnp.int32
    wid = lax.axis_index("s") * i32(_NC) + lax.axis_index("c")
    base = wid * i32(_PER_W)

    zi = jnp.zeros((_L,), jnp.int32)
    for k in range(_NB):
        acc_lab[pl.ds(k * _L, _L)] = zi

    def start(i, bbuf, qbuf, sem):
        off = base + i * i32(_C)
        offh = wid * i32(_PER_W // 2) + i * i32(_C // 2)
        pltpu.async_copy(labels_hbm.at[pl.ds(off, _C)], bbuf, sem)
        pltpu.async_copy(bins_hbm.at[pl.ds(offh, _C // 2)], qbuf, sem)

    def wait(bbuf, qbuf, sem):
        pltpu.make_async_copy(labels_hbm.at[pl.ds(0, _C)], bbuf, sem).wait()
        pltpu.make_async_copy(bins_hbm.at[pl.ds(0, _C // 2)],
                              qbuf, sem).wait()

    start(0, bbuf0, qbuf0, sem0)
    start(1, bbuf1, qbuf1, sem1)

    lane = lax.iota(jnp.int32, _L)

    def consume(bbuf, qbuf):
        def inner(j, carry):
            base_e = j * i32(_U * _L)
            offs = [base_e + i32(u * _L) for u in range(_U)]
            labs = [plsc.bitcast(bbuf[pl.ds(o, _L)], jnp.int32)
                    for o in offs]
            base_h = j * i32(_U * _L // 2)
            addrs = []
            for g in range(_U // 2):
                w = qbuf[pl.ds(base_h + i32(g * _L), _L)]
                a0, a1 = plsc.unpack(plsc.bitcast(w, jnp.int16),
                                     format=plsc.PackFormat.INTERLEAVED)
                addrs += [a0, a1]
            for u in range(_U):
                plsc.addupdate_scatter(acc_lab, [addrs[u]], labs[u])
            return carry
        lax.fori_loop(i32(0), i32(_C // (_U * _L)), inner, i32(0))

    def outer(t, carry):
        i0 = t * i32(2)
        wait(bbuf0, qbuf0, sem0)
        consume(bbuf0, qbuf0)

        @pl.when(i0 + i32(2) < i32(_NCHUNK))
        def _():
            start(i0 + i32(2), bbuf0, qbuf0, sem0)

        wait(bbuf1, qbuf1, sem1)
        consume(bbuf1, qbuf1)

        @pl.when(i0 + i32(3) < i32(_NCHUNK))
        def _():
            start(i0 + i32(3), bbuf1, qbuf1, sem1)

        return carry

    lax.fori_loop(i32(0), i32(_NCHUNK // 2), outer, i32(0))

    pltpu.sync_copy(acc_lab, outi_hbm.at[pl.ds(wid * i32(_ACC), _ACC)])


_MESH = plsc.VectorSubcoreMesh(core_axis_name="c", subcore_axis_name="s",
                               num_cores=_NC, num_subcores=_NS)


@jax.jit
def _sc_run(logits, labels_u32, tbl_base, tbl_slope):
    pass_a = pl.kernel(
        _pass_a_body,
        out_type=(
            jax.ShapeDtypeStruct((_NW * 2 * _ACC,), jnp.float32),
            jax.ShapeDtypeStruct((_N // 2,), jnp.int32),
        ),
        mesh=_MESH,
        scratch_types=[
            pltpu.VMEM((_C,), jnp.float32),
            pltpu.VMEM((_C,), jnp.float32),
            pltpu.VMEM((_C // 2,), jnp.int32),
            pltpu.VMEM((_C // 2,), jnp.int32),
            pltpu.VMEM((_TBL_N,), jnp.float32),
            pltpu.VMEM((_TBL_N,), jnp.float32),
            pltpu.VMEM((_ACC,), jnp.float32),
            pltpu.VMEM((_ACC,), jnp.float32),
            pltpu.SemaphoreType.DMA,
            pltpu.SemaphoreType.DMA,
            pltpu.SemaphoreType.DMA,
            pltpu.SemaphoreType.DMA,
        ],
        compiler_params=pltpu.CompilerParams(needs_layout_passes=False),
    )
    outf, bins = pass_a(logits, tbl_base, tbl_slope)

    pass_b = pl.kernel(
        _pass_b_body,
        out_type=jax.ShapeDtypeStruct((_NW * _ACC,), jnp.int32),
        mesh=_MESH,
        scratch_types=[
            pltpu.VMEM((_C,), jnp.uint32),
            pltpu.VMEM((_C,), jnp.uint32),
            pltpu.VMEM((_C // 2,), jnp.int32),
            pltpu.VMEM((_C // 2,), jnp.int32),
            pltpu.VMEM((_ACC,), jnp.int32),
            pltpu.SemaphoreType.DMA,
            pltpu.SemaphoreType.DMA,
        ],
        compiler_params=pltpu.CompilerParams(needs_layout_passes=False),
    )
    outi = pass_b(labels_u32, bins)
    return outf, outi


def _combine_body(pf_ref, pi_ref, pc_ref, e_ref, m_ref):
    xf = pf_ref[...]
    xi = pi_ref[...]
    sf = jnp.sum(xf, axis=(0, 3))
    lab = jnp.sum(xi.astype(jnp.float32), axis=(0, 2))
    cnt = sf[0]
    cf = sf[1]
    nonempty = cnt > 0.0
    denom = jnp.maximum(cnt, 1.0)
    pos = jnp.where(nonempty, lab / denom, 0.0)
    cfm = jnp.where(nonempty, cf / denom, 0.0)
    ece_i = jnp.abs(pos - cfm)
    pc_ref[...] = jnp.stack([pos, cfm])
    e_ref[...] = jnp.sum(ece_i).reshape(1, 1)
    m_ref[...] = jnp.max(ece_i).reshape(1, 1)


def kernel(logits, labels):
    labels_u32 = labels.astype(jnp.uint32)
    partials_f, partials_i = _sc_run(
        logits, labels_u32, jnp.asarray(_TBL_BASE), jnp.asarray(_TBL_SLOPE))
    pf = partials_f.reshape(_NW, 2, _NB, _L)
    pi = partials_i.reshape(_NW, _NB, _L)
    pc, e, m = pl.pallas_call(
        _combine_body,
        out_shape=[
            jax.ShapeDtypeStruct((2, _NB), jnp.float32),
            jax.ShapeDtypeStruct((1, 1), jnp.float32),
            jax.ShapeDtypeStruct((1, 1), jnp.float32),
        ],
    )(pf, pi)
    return (pc[0], pc[1], e[0, 0], m[0, 0])
